# baseline (speedup 1.0000x reference)
import functools

import jax
import jax.numpy as jnp
from jax import lax
from jax.experimental import pallas as pl
from jax.experimental.pallas import tpu as pltpu
from jax.experimental.pallas import tpu_sc as plsc

N = 10000
E = 320000
NFEAT = 128
D1 = 64
D2 = 16
NCLASS = 40
D3 = 48

NC = 2
NS = 16
NW = NC * NS
EPW = E // NW
CHUNK = 80
NCH = EPW // CHUNK
BC = 400
SS = BC // CHUNK
NCHB = EPW // BC
SUPB = 5
NB = 2
RPT = N // NS
RB = N // 1000


def _sc_mesh():
    return plsc.VectorSubcoreMesh(core_axis_name="c", subcore_axis_name="s")


def _make_deg_kernel():
    @functools.partial(
        pl.kernel,
        out_type=jax.ShapeDtypeStruct((RB, NW, 1000), jnp.float32),
        mesh=_sc_mesh(),
        scratch_types=[
            pltpu.VMEM((EPW,), jnp.int32),
            pltpu.VMEM((EPW,), jnp.float32),
            pltpu.VMEM((N,), jnp.float32),
        ],
        compiler_params=pltpu.CompilerParams(needs_layout_passes=False, use_tc_tiling_on_sc=False),
    )
    def deg_k(col_hbm, w_hbm, out_hbm, cbuf, wbuf, acc):
        cid = lax.axis_index("c")
        sid = lax.axis_index("s")
        wid = sid * NC + cid
        z = jnp.zeros((16,), jnp.float32)
        for j in range(N // 16):
            acc[pl.ds(j * 16, 16)] = z
        pltpu.sync_copy(col_hbm.at[wid], cbuf)
        pltpu.sync_copy(w_hbm.at[wid], wbuf)

        def body(i, carry):
            for g in range(CHUNK // 16):
                idx = cbuf[pl.ds(i * CHUNK + g * 16, 16)]
                val = wbuf[pl.ds(i * CHUNK + g * 16, 16)]
                plsc.addupdate_scatter(acc, [idx], val)
            return carry

        lax.fori_loop(0, NCH, body, 0)
        for j in range(RB):
            pltpu.sync_copy(acc.at[pl.ds(j * 1000, 1000)], out_hbm.at[j, wid])

    return deg_k


def _make_agg_kernel(D):
    @functools.partial(
        pl.kernel,
        out_type=jax.ShapeDtypeStruct((NC, N, D), jnp.float32),
        mesh=_sc_mesh(),
        scratch_types=[
            pltpu.VMEM((EPW,), jnp.int32),
            pltpu.VMEM((NCH, CHUNK), jnp.int32),
            pltpu.VMEM((EPW,), jnp.float32),
            pltpu.VMEM((NB, BC, D), jnp.float32),
            pltpu.VMEM_SHARED((N, D), jnp.float32),
            pltpu.SemaphoreType.DMA((NB,)),
            pltpu.SemaphoreType.DMA((NB, SS)),
        ],
        compiler_params=pltpu.CompilerParams(needs_layout_passes=False, use_tc_tiling_on_sc=False),
    )
    def agg_k(row_hbm, col_hbm, w_hbm, g_hbm, out_hbm, ridx, cidx, wbuf,
              rows, acc, gsem, ssem):
        cid = lax.axis_index("c")
        sid = lax.axis_index("s")
        wid = sid * NC + cid
        r0 = sid * RPT
        pltpu.sync_copy(g_hbm.at[pl.ds(r0, RPT)], acc.at[pl.ds(r0, RPT)])
        pltpu.sync_copy(row_hbm.at[wid], ridx)
        pltpu.sync_copy(col_hbm.at[wid], cidx)
        pltpu.sync_copy(w_hbm.at[wid], wbuf)
        plsc.subcore_barrier()

        def body(i, carry):
            base = i * SUPB
            gd = {}
            sd = {}
            for c in range(SUPB + 1):
                if c < SUPB:
                    b = c % NB
                    if c >= NB:
                        for s in range(SS):
                            sd[c - NB][s].wait()
                    gd[c] = pltpu.async_copy(
                        g_hbm.at[ridx.at[pl.ds((base + c) * BC, BC)]],
                        rows.at[b], gsem.at[b])
                pc = c - 1
                if pc >= 0:
                    pb = pc % NB
                    gd[pc].wait()
                    iv = jnp.full((16,), (base + pc) * BC, jnp.int32)

                    @plsc.parallel_loop(0, BC, unroll=8)
                    def _(e, _b=pb, _iv=iv):
                        ev = _iv + jnp.full((16,), e, jnp.int32)
                        nbv = plsc.load_gather(wbuf, [ev])
                        for f in range(D // 16):
                            sl = pl.ds(f * 16, 16)
                            rows[_b, e, sl] = rows[_b, e, sl] * nbv

                    sd[pc] = [
                        pltpu.async_copy(
                            rows.at[pb, pl.ds(s * CHUNK, CHUNK)],
                            acc.at[cidx.at[(base + pc) * SS + s]],
                            ssem.at[pb, s], add=True)
                        for s in range(SS)
                    ]
            for c in range(SUPB - NB, SUPB):
                for s in range(SS):
                    sd[c][s].wait()
            return carry

        lax.fori_loop(0, NCHB // SUPB, body, 0)
        plsc.subcore_barrier()
        pltpu.sync_copy(acc.at[pl.ds(r0, RPT)], out_hbm.at[cid, pl.ds(r0, RPT)])

    return agg_k


def _tc0(x, W1):
    def body(x_ref, w1_ref, h_ref):
        h_ref[...] = jnp.dot(x_ref[...], w1_ref[...],
                             preferred_element_type=jnp.float32)

    return pl.pallas_call(
        body,
        grid=(RB,),
        in_specs=[
            pl.BlockSpec((1000, NFEAT), lambda i: (i, 0)),
            pl.BlockSpec((NFEAT, D1), lambda i: (0, 0)),
        ],
        out_specs=pl.BlockSpec((1000, D1), lambda i: (i, 0)),
        out_shape=jax.ShapeDtypeStruct((N, D1), jnp.float32),
    )(x, W1)


def _tc1(degP, h):
    def body(deg_ref, h_ref, dis_ref, g1_ref):
        deg = jnp.sum(deg_ref[...], axis=(0, 1)) + 1.0
        dis = lax.rsqrt(deg)[:, None]
        dis_ref[...] = dis
        g1_ref[...] = h_ref[...] * dis

    return pl.pallas_call(
        body,
        grid=(RB,),
        in_specs=[
            pl.BlockSpec((1, NW, 1000), lambda i: (i, 0, 0)),
            pl.BlockSpec((1000, D1), lambda i: (i, 0)),
        ],
        out_specs=[
            pl.BlockSpec((1000, 1), lambda i: (i, 0)),
            pl.BlockSpec((1000, D1), lambda i: (i, 0)),
        ],
        out_shape=[
            jax.ShapeDtypeStruct((N, 1), jnp.float32),
            jax.ShapeDtypeStruct((N, D1), jnp.float32),
        ],
    )(degP, h)


def _tc_mid(p, g, dis, b, W, Din, Dout):
    def body(p_ref, g_ref, dis_ref, b_ref, w_ref, o_ref):
        pv = p_ref[...]
        agg = pv[0] + pv[1] - g_ref[...]
        pre = dis_ref[...] * agg + b_ref[...]
        a = jnp.maximum(pre, 0.0)
        h = jnp.dot(a, w_ref[...], preferred_element_type=jnp.float32)
        o_ref[...] = h * dis_ref[...]

    return pl.pallas_call(
        body,
        grid=(RB,),
        in_specs=[
            pl.BlockSpec((NC, 1000, Din), lambda i: (0, i, 0)),
            pl.BlockSpec((1000, Din), lambda i: (i, 0)),
            pl.BlockSpec((1000, 1), lambda i: (i, 0)),
            pl.BlockSpec((1, Din), lambda i: (0, 0)),
            pl.BlockSpec((Din, Dout), lambda i: (0, 0)),
        ],
        out_specs=pl.BlockSpec((1000, Dout), lambda i: (i, 0)),
        out_shape=jax.ShapeDtypeStruct((N, Dout), jnp.float32),
    )(p, g, dis, b, W)


def _tc_final(p, g, dis, b):
    def body(p_ref, g_ref, dis_ref, b_ref, o_ref):
        pv = p_ref[...]
        v = dis_ref[...] * (pv[0] + pv[1] - g_ref[...]) + b_ref[...]
        logits = v[:, :NCLASS]
        m = jnp.max(logits, axis=1, keepdims=True)
        ex = jnp.exp(logits - m)
        lse = jnp.log(jnp.sum(ex, axis=1, keepdims=True)) + m
        o_ref[...] = logits - lse

    return pl.pallas_call(
        body,
        grid=(RB,),
        in_specs=[
            pl.BlockSpec((NC, 1000, D3), lambda i: (0, i, 0)),
            pl.BlockSpec((1000, D3), lambda i: (i, 0)),
            pl.BlockSpec((1000, 1), lambda i: (i, 0)),
            pl.BlockSpec((1, D3), lambda i: (0, 0)),
        ],
        out_specs=pl.BlockSpec((1000, NCLASS), lambda i: (i, 0)),
        out_shape=jax.ShapeDtypeStruct((N, NCLASS), jnp.float32),
    )(p, g, dis, b)


def kernel(x, adj_indices, adj_values, W1, b1, W2, b2, W3, b3):
    row = adj_indices[0].astype(jnp.int32).reshape(NW, EPW)
    col = adj_indices[1].astype(jnp.int32).reshape(NW, NCH, CHUNK)
    w = adj_values.reshape(NW, EPW)
    W3p = jnp.pad(W3, ((0, 0), (0, D3 - NCLASS)))
    b3p = jnp.pad(b3, (0, D3 - NCLASS)).reshape(1, D3)
    b1r = b1.reshape(1, D1)
    b2r = b2.reshape(1, D2)

    colf = adj_indices[1].astype(jnp.int32).reshape(NW, EPW)
    h1 = _tc0(x, W1)
    degP = _make_deg_kernel()(colf, w)
    dis, g1 = _tc1(degP, h1)
    p1 = _make_agg_kernel(D1)(row, col, w, g1)
    g2 = _tc_mid(p1, g1, dis, b1r, W2, D1, D2)
    p2 = _make_agg_kernel(D2)(row, col, w, g2)
    g3 = _tc_mid(p2, g2, dis, b2r, W3p, D2, D3)
    p3 = _make_agg_kernel(D3)(row, col, w, g3)
    return _tc_final(p3, g3, dis, b3p)


# speedup vs baseline: 1.0493x; 1.0493x over previous
import functools

import jax
import jax.numpy as jnp
from jax import lax
from jax.experimental import pallas as pl
from jax.experimental.pallas import tpu as pltpu
from jax.experimental.pallas import tpu_sc as plsc

N = 10000
E = 320000
NFEAT = 128
D1 = 64
D2 = 16
NCLASS = 40
D3 = 48

NC = 2
NS = 16
NW = NC * NS
EPW = E // NW
CHUNK = 80
NCH = EPW // CHUNK
SUP = 25
NB = 6
LKA = 3
RPT = N // NS
RB = N // 1000


def _sc_mesh():
    return plsc.VectorSubcoreMesh(core_axis_name="c", subcore_axis_name="s")


def _make_deg_kernel():
    @functools.partial(
        pl.kernel,
        out_type=jax.ShapeDtypeStruct((RB, NW, 1000), jnp.float32),
        mesh=_sc_mesh(),
        scratch_types=[
            pltpu.VMEM((EPW,), jnp.int32),
            pltpu.VMEM((EPW,), jnp.float32),
            pltpu.VMEM((N,), jnp.float32),
        ],
        compiler_params=pltpu.CompilerParams(needs_layout_passes=False, use_tc_tiling_on_sc=False),
    )
    def deg_k(col_hbm, w_hbm, out_hbm, cbuf, wbuf, acc):
        cid = lax.axis_index("c")
        sid = lax.axis_index("s")
        wid = sid * NC + cid
        z = jnp.zeros((16,), jnp.float32)
        for j in range(N // 16):
            acc[pl.ds(j * 16, 16)] = z
        pltpu.sync_copy(col_hbm.at[wid], cbuf)
        pltpu.sync_copy(w_hbm.at[wid], wbuf)

        def body(i, carry):
            for g in range(CHUNK // 16):
                idx = cbuf[pl.ds(i * CHUNK + g * 16, 16)]
                val = wbuf[pl.ds(i * CHUNK + g * 16, 16)]
                plsc.addupdate_scatter(acc, [idx], val)
            return carry

        lax.fori_loop(0, NCH, body, 0)
        for j in range(RB):
            pltpu.sync_copy(acc.at[pl.ds(j * 1000, 1000)], out_hbm.at[j, wid])

    return deg_k


def _make_agg_kernel(D):
    @functools.partial(
        pl.kernel,
        out_type=jax.ShapeDtypeStruct((NC, N, D), jnp.float32),
        mesh=_sc_mesh(),
        scratch_types=[
            pltpu.VMEM((EPW,), jnp.int32),
            pltpu.VMEM((NCH, CHUNK), jnp.int32),
            pltpu.VMEM((EPW,), jnp.float32),
            pltpu.VMEM((NB, CHUNK, D), jnp.float32),
            pltpu.VMEM_SHARED((N, D), jnp.float32),
            pltpu.SemaphoreType.DMA((NB,)),
            pltpu.SemaphoreType.DMA((NB,)),
        ],
        compiler_params=pltpu.CompilerParams(needs_layout_passes=False, use_tc_tiling_on_sc=False),
    )
    def agg_k(row_hbm, col_hbm, w_hbm, g_hbm, out_hbm, ridx, cidx, wbuf,
              rows, acc, gsem, ssem):
        cid = lax.axis_index("c")
        sid = lax.axis_index("s")
        wid = sid * NC + cid
        r0 = sid * RPT
        pltpu.sync_copy(g_hbm.at[pl.ds(r0, RPT)], acc.at[pl.ds(r0, RPT)])
        pltpu.sync_copy(row_hbm.at[wid], ridx)
        pltpu.sync_copy(col_hbm.at[wid], cidx)
        pltpu.sync_copy(w_hbm.at[wid], wbuf)
        plsc.subcore_barrier()

        def body(i, carry):
            base = i * SUP
            gd = {}
            sd = {}
            for c in range(SUP + LKA):
                if c < SUP:
                    b = c % NB
                    if c >= NB:
                        sd[c - NB].wait()
                    gd[c] = pltpu.async_copy(
                        g_hbm.at[ridx.at[pl.ds((base + c) * CHUNK, CHUNK)]],
                        rows.at[b], gsem.at[b])
                pc = c - LKA
                if pc >= 0:
                    pb = pc % NB
                    gd[pc].wait()
                    iv = jnp.full((16,), (base + pc) * CHUNK, jnp.int32)

                    @plsc.parallel_loop(0, CHUNK, unroll=8)
                    def _(e, _b=pb, _iv=iv):
                        ev = _iv + jnp.full((16,), e, jnp.int32)
                        nbv = plsc.load_gather(wbuf, [ev])
                        for f in range(D // 16):
                            sl = pl.ds(f * 16, 16)
                            rows[_b, e, sl] = rows[_b, e, sl] * nbv

                    sd[pc] = pltpu.async_copy(rows.at[pb],
                                              acc.at[cidx.at[base + pc]],
                                              ssem.at[pb], add=True)
            for c in range(SUP - NB, SUP):
                sd[c].wait()
            return carry

        lax.fori_loop(0, NCH // SUP, body, 0)
        plsc.subcore_barrier()
        pltpu.sync_copy(acc.at[pl.ds(r0, RPT)], out_hbm.at[cid, pl.ds(r0, RPT)])

    return agg_k


def _tc0(x, W1):
    def body(x_ref, w1_ref, h_ref):
        h_ref[...] = jnp.dot(x_ref[...], w1_ref[...],
                             preferred_element_type=jnp.float32)

    return pl.pallas_call(
        body,
        grid=(RB,),
        in_specs=[
            pl.BlockSpec((1000, NFEAT), lambda i: (i, 0)),
            pl.BlockSpec((NFEAT, D1), lambda i: (0, 0)),
        ],
        out_specs=pl.BlockSpec((1000, D1), lambda i: (i, 0)),
        out_shape=jax.ShapeDtypeStruct((N, D1), jnp.float32),
    )(x, W1)


def _tc1(degP, h):
    def body(deg_ref, h_ref, dis_ref, g1_ref):
        deg = jnp.sum(deg_ref[...], axis=(0, 1)) + 1.0
        dis = lax.rsqrt(deg)[:, None]
        dis_ref[...] = dis
        g1_ref[...] = h_ref[...] * dis

    return pl.pallas_call(
        body,
        grid=(RB,),
        in_specs=[
            pl.BlockSpec((1, NW, 1000), lambda i: (i, 0, 0)),
            pl.BlockSpec((1000, D1), lambda i: (i, 0)),
        ],
        out_specs=[
            pl.BlockSpec((1000, 1), lambda i: (i, 0)),
            pl.BlockSpec((1000, D1), lambda i: (i, 0)),
        ],
        out_shape=[
            jax.ShapeDtypeStruct((N, 1), jnp.float32),
            jax.ShapeDtypeStruct((N, D1), jnp.float32),
        ],
    )(degP, h)


def _tc_mid(p, g, dis, b, W, Din, Dout):
    def body(p_ref, g_ref, dis_ref, b_ref, w_ref, o_ref):
        pv = p_ref[...]
        agg = pv[0] + pv[1] - g_ref[...]
        pre = dis_ref[...] * agg + b_ref[...]
        a = jnp.maximum(pre, 0.0)
        h = jnp.dot(a, w_ref[...], preferred_element_type=jnp.float32)
        o_ref[...] = h * dis_ref[...]

    return pl.pallas_call(
        body,
        grid=(RB,),
        in_specs=[
            pl.BlockSpec((NC, 1000, Din), lambda i: (0, i, 0)),
            pl.BlockSpec((1000, Din), lambda i: (i, 0)),
            pl.BlockSpec((1000, 1), lambda i: (i, 0)),
            pl.BlockSpec((1, Din), lambda i: (0, 0)),
            pl.BlockSpec((Din, Dout), lambda i: (0, 0)),
        ],
        out_specs=pl.BlockSpec((1000, Dout), lambda i: (i, 0)),
        out_shape=jax.ShapeDtypeStruct((N, Dout), jnp.float32),
    )(p, g, dis, b, W)


def _tc_final(p, g, dis, b):
    def body(p_ref, g_ref, dis_ref, b_ref, o_ref):
        pv = p_ref[...]
        v = dis_ref[...] * (pv[0] + pv[1] - g_ref[...]) + b_ref[...]
        logits = v[:, :NCLASS]
        m = jnp.max(logits, axis=1, keepdims=True)
        ex = jnp.exp(logits - m)
        lse = jnp.log(jnp.sum(ex, axis=1, keepdims=True)) + m
        o_ref[...] = logits - lse

    return pl.pallas_call(
        body,
        grid=(RB,),
        in_specs=[
            pl.BlockSpec((NC, 1000, D3), lambda i: (0, i, 0)),
            pl.BlockSpec((1000, D3), lambda i: (i, 0)),
            pl.BlockSpec((1000, 1), lambda i: (i, 0)),
            pl.BlockSpec((1, D3), lambda i: (0, 0)),
        ],
        out_specs=pl.BlockSpec((1000, NCLASS), lambda i: (i, 0)),
        out_shape=jax.ShapeDtypeStruct((N, NCLASS), jnp.float32),
    )(p, g, dis, b)


def kernel(x, adj_indices, adj_values, W1, b1, W2, b2, W3, b3):
    row = adj_indices[0].astype(jnp.int32).reshape(NW, EPW)
    col = adj_indices[1].astype(jnp.int32).reshape(NW, NCH, CHUNK)
    w = adj_values.reshape(NW, EPW)
    W3p = jnp.pad(W3, ((0, 0), (0, D3 - NCLASS)))
    b3p = jnp.pad(b3, (0, D3 - NCLASS)).reshape(1, D3)
    b1r = b1.reshape(1, D1)
    b2r = b2.reshape(1, D2)

    colf = adj_indices[1].astype(jnp.int32).reshape(NW, EPW)
    h1 = _tc0(x, W1)
    degP = _make_deg_kernel()(colf, w)
    dis, g1 = _tc1(degP, h1)
    p1 = _make_agg_kernel(D1)(row, col, w, g1)
    g2 = _tc_mid(p1, g1, dis, b1r, W2, D1, D2)
    p2 = _make_agg_kernel(D2)(row, col, w, g2)
    g3 = _tc_mid(p2, g2, dis, b2r, W3p, D2, D3)
    p3 = _make_agg_kernel(D3)(row, col, w, g3)
    return _tc_final(p3, g3, dis, b3p)


# speedup vs baseline: 1.0697x; 1.0194x over previous
import functools

import jax
import jax.numpy as jnp
from jax import lax
from jax.experimental import pallas as pl
from jax.experimental.pallas import tpu as pltpu
from jax.experimental.pallas import tpu_sc as plsc

N = 10000
E = 320000
NFEAT = 128
D1 = 64
D2 = 16
NCLASS = 40
D3 = 48

NC = 2
NS = 16
NW = NC * NS
EPW = E // NW
CHUNK = 80
NCH = EPW // CHUNK
SUP = 25
NB = 8
LKA = 4
RPT = N // NS
RB = N // 1000


def _sc_mesh():
    return plsc.VectorSubcoreMesh(core_axis_name="c", subcore_axis_name="s")


def _make_deg_kernel():
    @functools.partial(
        pl.kernel,
        out_type=jax.ShapeDtypeStruct((RB, NW, 1000), jnp.float32),
        mesh=_sc_mesh(),
        scratch_types=[
            pltpu.VMEM((EPW,), jnp.int32),
            pltpu.VMEM((EPW,), jnp.float32),
            pltpu.VMEM((N,), jnp.float32),
        ],
        compiler_params=pltpu.CompilerParams(needs_layout_passes=False, use_tc_tiling_on_sc=False),
    )
    def deg_k(col_hbm, w_hbm, out_hbm, cbuf, wbuf, acc):
        cid = lax.axis_index("c")
        sid = lax.axis_index("s")
        wid = sid * NC + cid
        z = jnp.zeros((16,), jnp.float32)
        for j in range(N // 16):
            acc[pl.ds(j * 16, 16)] = z
        pltpu.sync_copy(col_hbm.at[wid], cbuf)
        pltpu.sync_copy(w_hbm.at[wid], wbuf)

        def body(i, carry):
            for g in range(CHUNK // 16):
                idx = cbuf[pl.ds(i * CHUNK + g * 16, 16)]
                val = wbuf[pl.ds(i * CHUNK + g * 16, 16)]
                plsc.addupdate_scatter(acc, [idx], val)
            return carry

        lax.fori_loop(0, NCH, body, 0)
        for j in range(RB):
            pltpu.sync_copy(acc.at[pl.ds(j * 1000, 1000)], out_hbm.at[j, wid])

    return deg_k


def _make_agg_kernel(D):
    @functools.partial(
        pl.kernel,
        out_type=jax.ShapeDtypeStruct((NC, N, D), jnp.float32),
        mesh=_sc_mesh(),
        scratch_types=[
            pltpu.VMEM((EPW,), jnp.int32),
            pltpu.VMEM((NCH, CHUNK), jnp.int32),
            pltpu.VMEM((EPW,), jnp.float32),
            pltpu.VMEM((NB, CHUNK, D), jnp.float32),
            pltpu.VMEM_SHARED((N, D), jnp.float32),
            pltpu.SemaphoreType.DMA((NB,)),
            pltpu.SemaphoreType.DMA((NB,)),
        ],
        compiler_params=pltpu.CompilerParams(needs_layout_passes=False, use_tc_tiling_on_sc=False),
    )
    def agg_k(row_hbm, col_hbm, w_hbm, g_hbm, out_hbm, ridx, cidx, wbuf,
              rows, acc, gsem, ssem):
        cid = lax.axis_index("c")
        sid = lax.axis_index("s")
        wid = sid * NC + cid
        r0 = sid * RPT
        pltpu.sync_copy(g_hbm.at[pl.ds(r0, RPT)], acc.at[pl.ds(r0, RPT)])
        pltpu.sync_copy(row_hbm.at[wid], ridx)
        pltpu.sync_copy(col_hbm.at[wid], cidx)
        pltpu.sync_copy(w_hbm.at[wid], wbuf)
        plsc.subcore_barrier()

        def body(i, carry):
            base = i * SUP
            gd = {}
            sd = {}
            for c in range(SUP + LKA):
                if c < SUP:
                    b = c % NB
                    if c >= NB:
                        sd[c - NB].wait()
                    gd[c] = pltpu.async_copy(
                        g_hbm.at[ridx.at[pl.ds((base + c) * CHUNK, CHUNK)]],
                        rows.at[b], gsem.at[b])
                pc = c - LKA
                if pc >= 0:
                    pb = pc % NB
                    gd[pc].wait()
                    iv = jnp.full((16,), (base + pc) * CHUNK, jnp.int32)

                    @plsc.parallel_loop(0, CHUNK, unroll=8)
                    def _(e, _b=pb, _iv=iv):
                        ev = _iv + jnp.full((16,), e, jnp.int32)
                        nbv = plsc.load_gather(wbuf, [ev])
                        for f in range(D // 16):
                            sl = pl.ds(f * 16, 16)
                            rows[_b, e, sl] = rows[_b, e, sl] * nbv

                    sd[pc] = pltpu.async_copy(rows.at[pb],
                                              acc.at[cidx.at[base + pc]],
                                              ssem.at[pb], add=True)
            for c in range(SUP - NB, SUP):
                sd[c].wait()
            return carry

        lax.fori_loop(0, NCH // SUP, body, 0)
        plsc.subcore_barrier()
        pltpu.sync_copy(acc.at[pl.ds(r0, RPT)], out_hbm.at[cid, pl.ds(r0, RPT)])

    return agg_k


def _tc0(x, W1):
    def body(x_ref, w1_ref, h_ref):
        h_ref[...] = jnp.dot(x_ref[...], w1_ref[...],
                             preferred_element_type=jnp.float32)

    return pl.pallas_call(
        body,
        grid=(RB,),
        in_specs=[
            pl.BlockSpec((1000, NFEAT), lambda i: (i, 0)),
            pl.BlockSpec((NFEAT, D1), lambda i: (0, 0)),
        ],
        out_specs=pl.BlockSpec((1000, D1), lambda i: (i, 0)),
        out_shape=jax.ShapeDtypeStruct((N, D1), jnp.float32),
    )(x, W1)


def _tc1(degP, h):
    def body(deg_ref, h_ref, dis_ref, g1_ref):
        deg = jnp.sum(deg_ref[...], axis=(0, 1)) + 1.0
        dis = lax.rsqrt(deg)[:, None]
        dis_ref[...] = dis
        g1_ref[...] = h_ref[...] * dis

    return pl.pallas_call(
        body,
        grid=(RB,),
        in_specs=[
            pl.BlockSpec((1, NW, 1000), lambda i: (i, 0, 0)),
            pl.BlockSpec((1000, D1), lambda i: (i, 0)),
        ],
        out_specs=[
            pl.BlockSpec((1000, 1), lambda i: (i, 0)),
            pl.BlockSpec((1000, D1), lambda i: (i, 0)),
        ],
        out_shape=[
            jax.ShapeDtypeStruct((N, 1), jnp.float32),
            jax.ShapeDtypeStruct((N, D1), jnp.float32),
        ],
    )(degP, h)


def _tc_mid(p, g, dis, b, W, Din, Dout):
    def body(p_ref, g_ref, dis_ref, b_ref, w_ref, o_ref):
        pv = p_ref[...]
        agg = pv[0] + pv[1] - g_ref[...]
        pre = dis_ref[...] * agg + b_ref[...]
        a = jnp.maximum(pre, 0.0)
        h = jnp.dot(a, w_ref[...], preferred_element_type=jnp.float32)
        o_ref[...] = h * dis_ref[...]

    return pl.pallas_call(
        body,
        grid=(RB,),
        in_specs=[
            pl.BlockSpec((NC, 1000, Din), lambda i: (0, i, 0)),
            pl.BlockSpec((1000, Din), lambda i: (i, 0)),
            pl.BlockSpec((1000, 1), lambda i: (i, 0)),
            pl.BlockSpec((1, Din), lambda i: (0, 0)),
            pl.BlockSpec((Din, Dout), lambda i: (0, 0)),
        ],
        out_specs=pl.BlockSpec((1000, Dout), lambda i: (i, 0)),
        out_shape=jax.ShapeDtypeStruct((N, Dout), jnp.float32),
    )(p, g, dis, b, W)


def _tc_final(p, g, dis, b):
    def body(p_ref, g_ref, dis_ref, b_ref, o_ref):
        pv = p_ref[...]
        v = dis_ref[...] * (pv[0] + pv[1] - g_ref[...]) + b_ref[...]
        logits = v[:, :NCLASS]
        m = jnp.max(logits, axis=1, keepdims=True)
        ex = jnp.exp(logits - m)
        lse = jnp.log(jnp.sum(ex, axis=1, keepdims=True)) + m
        o_ref[...] = logits - lse

    return pl.pallas_call(
        body,
        grid=(RB,),
        in_specs=[
            pl.BlockSpec((NC, 1000, D3), lambda i: (0, i, 0)),
            pl.BlockSpec((1000, D3), lambda i: (i, 0)),
            pl.BlockSpec((1000, 1), lambda i: (i, 0)),
            pl.BlockSpec((1, D3), lambda i: (0, 0)),
        ],
        out_specs=pl.BlockSpec((1000, NCLASS), lambda i: (i, 0)),
        out_shape=jax.ShapeDtypeStruct((N, NCLASS), jnp.float32),
    )(p, g, dis, b)


def kernel(x, adj_indices, adj_values, W1, b1, W2, b2, W3, b3):
    row = adj_indices[0].astype(jnp.int32).reshape(NW, EPW)
    col = adj_indices[1].astype(jnp.int32).reshape(NW, NCH, CHUNK)
    w = adj_values.reshape(NW, EPW)
    W3p = jnp.pad(W3, ((0, 0), (0, D3 - NCLASS)))
    b3p = jnp.pad(b3, (0, D3 - NCLASS)).reshape(1, D3)
    b1r = b1.reshape(1, D1)
    b2r = b2.reshape(1, D2)

    colf = adj_indices[1].astype(jnp.int32).reshape(NW, EPW)
    h1 = _tc0(x, W1)
    degP = _make_deg_kernel()(colf, w)
    dis, g1 = _tc1(degP, h1)
    p1 = _make_agg_kernel(D1)(row, col, w, g1)
    g2 = _tc_mid(p1, g1, dis, b1r, W2, D1, D2)
    p2 = _make_agg_kernel(D2)(row, col, w, g2)
    g3 = _tc_mid(p2, g2, dis, b2r, W3p, D2, D3)
    p3 = _make_agg_kernel(D3)(row, col, w, g3)
    return _tc_final(p3, g3, dis, b3p)
